# Initial kernel scaffold; baseline (speedup 1.0000x reference)
import jax
import jax.numpy as jnp
from jax.experimental import pallas as pl


def kernel(x, weight, action, embed_e0):
    raise NotImplementedError("write your pallas kernel here")


# speedup vs baseline: 4.6916x; 4.6916x over previous
import itertools

import numpy as np
import jax
import jax.numpy as jnp
from jax.experimental import pallas as pl
from jax.experimental.pallas import tpu as pltpu


def _quad_structure():
    metric = [0, 1, 1, 1]
    blades = [()]
    for g in range(1, 5):
        blades += list(itertools.combinations(range(4), g))
    idx = {b: i for i, b in enumerate(blades)}
    M = np.zeros((16, 16, 16), np.float64)
    for i, a in enumerate(blades):
        for j, b in enumerate(blades):
            arr = list(a) + list(b)
            sign = 1
            n = len(arr)
            for p in range(n):
                for q in range(n - 1):
                    if arr[q] > arr[q + 1]:
                        arr[q], arr[q + 1] = arr[q + 1], arr[q]
                        sign = -sign
            res, k = [], 0
            while k < len(arr):
                if k + 1 < len(arr) and arr[k] == arr[k + 1]:
                    sign *= metric[arr[k]]
                    k += 2
                else:
                    res.append(arr[k])
                    k += 1
            if sign != 0:
                M[i, j, idx[tuple(res)]] += sign
    grades = np.array([len(b) for b in blades])
    rev = (-1.0) ** (grades * (grades - 1) // 2)

    ab = [0, 5, 6, 7, 8, 9, 10, 15]
    pout = [11, 12, 13]
    rin = [11, 12, 13, 14]
    Q = np.einsum('qpm,nqr->prmn', M[:, pout][:, :, ab], M[ab][:, :, rin])
    Q = Q * np.asarray(rev)[ab][None, None, :, None]
    Qs = 0.5 * (Q + Q.swapaxes(2, 3))

    def terms(p, r):
        out = []
        for s in range(8):
            for t in range(s, 8):
                c = Qs[p, r, s, t] * (2.0 if t != s else 1.0)
                if c != 0.0:
                    out.append((s, t, float(c)))
        return out

    diag = terms(0, 0)
    for p in range(3):
        assert terms(p, p) == diag
        for r in range(3):
            if r != p:
                assert not terms(p, r)
    g_terms = [terms(p, 3) for p in range(3)]
    return diag, g_terms


_DIAG_TERMS, _G_TERMS = _quad_structure()


def _accum(a, terms):
    acc = None
    for s, t, c in terms:
        prod = a[s] * a[t]
        if c == -1.0:
            prod = -prod
        elif c != 1.0:
            prod = c * prod
        acc = prod if acc is None else acc + prod
    return acc


def _build_kernel(at_ref, wt_ref, e0_ref, st_ref, b_ref):
    a = [at_ref[s] for s in range(8)]
    wt = wt_ref[...]
    st_ref[...] = wt * _accum(a, _DIAG_TERMS)
    we = wt * e0_ref[...]
    for p in range(3):
        gp = _accum(a, _G_TERMS[p])
        b_ref[p:p + 1, :] = jnp.sum(we * gp, axis=0, keepdims=True)
    b_ref[3:8, :] = jnp.zeros_like(b_ref[3:8, :])


def _mm_kernel(x_ref, w_ref, b_ref, o_ref):
    o_ref[...] = (
        jnp.dot(x_ref[...], w_ref[...], preferred_element_type=jnp.float32)
        + b_ref[...]
    )


def kernel(x, weight, action, embed_e0):
    B, I, _ = x.shape
    O = weight.shape[0]
    f32 = jnp.float32

    at = jnp.transpose(action, (2, 1, 0))
    wt = weight.T

    BO = 256 if O % 256 == 0 else O
    st, bias8 = pl.pallas_call(
        _build_kernel,
        grid=(O // BO,),
        in_specs=[
            pl.BlockSpec((8, I, BO), lambda j: (0, 0, j)),
            pl.BlockSpec((I, BO), lambda j: (0, j)),
            pl.BlockSpec((I, 1), lambda j: (0, 0)),
        ],
        out_specs=[
            pl.BlockSpec((I, BO), lambda j: (0, j)),
            pl.BlockSpec((8, BO), lambda j: (0, j)),
        ],
        out_shape=[
            jax.ShapeDtypeStruct((I, O), f32),
            jax.ShapeDtypeStruct((8, O), f32),
        ],
        compiler_params=pltpu.CompilerParams(
            dimension_semantics=("parallel",),
        ),
        name="pga_build_s",
    )(at, wt, embed_e0)

    eye3 = jnp.eye(3, dtype=f32)
    w2 = (st[:, None, :, None] * eye3[None, :, None, :]).reshape(3 * I, 3 * O)
    bias_flat = bias8[:3].T.reshape(1, 3 * O)

    BB = 512 if B % 512 == 0 else B
    out2 = pl.pallas_call(
        _mm_kernel,
        grid=(B // BB,),
        in_specs=[
            pl.BlockSpec((BB, 3 * I), lambda i: (i, 0)),
            pl.BlockSpec((3 * I, 3 * O), lambda i: (0, 0)),
            pl.BlockSpec((1, 3 * O), lambda i: (0, 0)),
        ],
        out_specs=pl.BlockSpec((BB, 3 * O), lambda i: (i, 0)),
        out_shape=jax.ShapeDtypeStruct((B, 3 * O), f32),
        compiler_params=pltpu.CompilerParams(
            dimension_semantics=("parallel",),
            vmem_limit_bytes=100 * 1024 * 1024,
        ),
        name="pga_matmul",
    )(x.reshape(B, 3 * I), w2, bias_flat)

    return out2.reshape(B, O, 3)
